# baseline (speedup 1.0000x reference)
import functools

import jax
import jax.numpy as jnp
from jax import lax
from jax.experimental import pallas as pl
from jax.experimental.pallas import tpu as pltpu
from jax.experimental.pallas import tpu_sc as plsc

BATCH = 4096
SEQ = 200
EMBED_DIM = 64
NUM_CORES = 2
NUM_SUBCORES = 16
NUM_WORKERS = NUM_CORES * NUM_SUBCORES
CHUNKS_PER_WORKER = BATCH // NUM_WORKERS
LANES = 16
VREGS_PER_ROW = EMBED_DIM // LANES
NBUF = 4
IDX_SPLITS = ((0, 128), (128, 72))


def _gather_descs(table_hbm, idx_v, buf, sem, k):
    return [
        pltpu.make_async_copy(
            table_hbm.at[idx_v.at[k, pl.ds(o, m)]], buf.at[pl.ds(o, m)], sem)
        for o, m in IDX_SPLITS
    ]


def _wb_desc(buf, out_hbm, sem, g_seq):
    return pltpu.make_async_copy(buf, out_hbm.at[g_seq], sem)


def _add_pos_pair(buf_a, buf_b, pos_v):
    def rows(r2, c):
        for dr in range(2):
            r = r2 * 2 + dr
            for j in range(VREGS_PER_ROW):
                sl = pl.ds(j * LANES, LANES)
                p = pos_v[r, sl]
                buf_a[r, sl] = buf_a[r, sl] + p
                buf_b[r, sl] = buf_b[r, sl] + p
        return c

    lax.fori_loop(0, SEQ // 2, rows, 0, unroll=4)


def _tpe_kernel(idx_hbm, table_hbm, pos_hbm, out_hbm,
                idx_v, pos_v, b0, b1, b2, b3, g0, g1, g2, g3, w0, w1, w2, w3):
    bufs = [b0, b1, b2, b3]
    gsems = [g0, g1, g2, g3]
    wsems = [w0, w1, w2, w3]
    wid = lax.axis_index("s") * NUM_CORES + lax.axis_index("c")
    seq_base = wid * CHUNKS_PER_WORKER
    n = CHUNKS_PER_WORKER

    pltpu.sync_copy(pos_hbm, pos_v)
    pltpu.sync_copy(idx_hbm.at[pl.ds(seq_base, n)], idx_v)

    for b in range(2):
        for d in _gather_descs(table_hbm, idx_v, bufs[b], gsems[b], b):
            d.start()

    def group(p, carry):
        for half in range(2):
            ba, bb = 2 * half, 2 * half + 1
            na, nb2 = (2 * half + 2) % NBUF, (2 * half + 3) % NBUF
            k = NBUF * p + 2 * half

            @pl.when(jnp.logical_and(k >= 2, k + 2 < n))
            def _():
                _wb_desc(bufs[na], out_hbm, wsems[na], seq_base + k - 2).wait()
                _wb_desc(bufs[nb2], out_hbm, wsems[nb2], seq_base + k - 1).wait()

            @pl.when(k + 2 < n)
            def _():
                for d in _gather_descs(table_hbm, idx_v, bufs[na], gsems[na],
                                       k + 2):
                    d.start()
                for d in _gather_descs(table_hbm, idx_v, bufs[nb2], gsems[nb2],
                                       k + 3):
                    d.start()

            for d in _gather_descs(table_hbm, idx_v, bufs[ba], gsems[ba], k):
                d.wait()
            for d in _gather_descs(table_hbm, idx_v, bufs[bb], gsems[bb], k + 1):
                d.wait()
            _add_pos_pair(bufs[ba], bufs[bb], pos_v)
            _wb_desc(bufs[ba], out_hbm, wsems[ba], seq_base + k).start()
            _wb_desc(bufs[bb], out_hbm, wsems[bb], seq_base + k + 1).start()
        return carry

    lax.fori_loop(0, n // NBUF, group, 0)

    for b in range(NBUF):
        _wb_desc(bufs[b], out_hbm, wsems[b],
                 seq_base + n - NBUF + b).wait()


def kernel(inputs, token_table, position_table):
    mesh = plsc.VectorSubcoreMesh(core_axis_name="c", subcore_axis_name="s")
    run = functools.partial(
        pl.kernel,
        out_type=jax.ShapeDtypeStruct((BATCH, SEQ, EMBED_DIM), jnp.float32),
        mesh=mesh,
        scratch_types=(
            [pltpu.VMEM((CHUNKS_PER_WORKER, SEQ), jnp.int32),
             pltpu.VMEM((SEQ, EMBED_DIM), jnp.float32)]
            + [pltpu.VMEM((SEQ, EMBED_DIM), jnp.float32) for _ in range(NBUF)]
            + [pltpu.SemaphoreType.DMA for _ in range(2 * NBUF)]
        ),
        compiler_params=pltpu.CompilerParams(use_tc_tiling_on_sc=False),
    )(_tpe_kernel)
    return run(inputs.astype(jnp.int32), token_table, position_table)


# speedup vs baseline: 1.3214x; 1.3214x over previous
import functools

import jax
import jax.numpy as jnp
from jax import lax
from jax.experimental import pallas as pl
from jax.experimental.pallas import tpu as pltpu
from jax.experimental.pallas import tpu_sc as plsc

BATCH = 4096
SEQ = 200
EMBED_DIM = 64
NUM_CORES = 2
NUM_SUBCORES = 16
NUM_WORKERS = NUM_CORES * NUM_SUBCORES
CHUNKS_PER_WORKER = BATCH // NUM_WORKERS
LANES = 16
VREGS_PER_ROW = EMBED_DIM // LANES
NBUF = 4
IDX_SPLITS = ((0, 128), (128, 72))


def _gather_descs(table_hbm, idx_v, buf, sem, k):
    return [
        pltpu.make_async_copy(
            table_hbm.at[idx_v.at[k, pl.ds(o, m)]], buf.at[pl.ds(o, m)], sem)
        for o, m in IDX_SPLITS
    ]


def _wb_desc(buf, out_hbm, sem, g_seq):
    return pltpu.make_async_copy(buf, out_hbm.at[g_seq], sem)


def _add_pos_pair(buf_a, buf_b, pos_v):
    @plsc.parallel_loop(0, SEQ, 1, unroll=4)
    def _row(r):
        for j in range(VREGS_PER_ROW):
            sl = pl.ds(j * LANES, LANES)
            p = pos_v[r, sl]
            buf_a[r, sl] = buf_a[r, sl] + p
            buf_b[r, sl] = buf_b[r, sl] + p


def _tpe_kernel(idx_hbm, table_hbm, pos_hbm, out_hbm,
                idx_v, pos_v, b0, b1, b2, b3, g0, g1, g2, g3, w0, w1, w2, w3):
    bufs = [b0, b1, b2, b3]
    gsems = [g0, g1, g2, g3]
    wsems = [w0, w1, w2, w3]
    wid = lax.axis_index("s") * NUM_CORES + lax.axis_index("c")
    seq_base = wid * CHUNKS_PER_WORKER
    n = CHUNKS_PER_WORKER

    pltpu.sync_copy(pos_hbm, pos_v)
    pltpu.sync_copy(idx_hbm.at[pl.ds(seq_base, n)], idx_v)

    for b in range(2):
        for d in _gather_descs(table_hbm, idx_v, bufs[b], gsems[b], b):
            d.start()

    def group(p, carry):
        for half in range(2):
            ba, bb = 2 * half, 2 * half + 1
            na, nb2 = (2 * half + 2) % NBUF, (2 * half + 3) % NBUF
            k = NBUF * p + 2 * half

            @pl.when(jnp.logical_and(k >= 2, k + 2 < n))
            def _():
                _wb_desc(bufs[na], out_hbm, wsems[na], seq_base + k - 2).wait()
                _wb_desc(bufs[nb2], out_hbm, wsems[nb2], seq_base + k - 1).wait()

            @pl.when(k + 2 < n)
            def _():
                for d in _gather_descs(table_hbm, idx_v, bufs[na], gsems[na],
                                       k + 2):
                    d.start()
                for d in _gather_descs(table_hbm, idx_v, bufs[nb2], gsems[nb2],
                                       k + 3):
                    d.start()

            for d in _gather_descs(table_hbm, idx_v, bufs[ba], gsems[ba], k):
                d.wait()
            for d in _gather_descs(table_hbm, idx_v, bufs[bb], gsems[bb], k + 1):
                d.wait()
            _add_pos_pair(bufs[ba], bufs[bb], pos_v)
            _wb_desc(bufs[ba], out_hbm, wsems[ba], seq_base + k).start()
            _wb_desc(bufs[bb], out_hbm, wsems[bb], seq_base + k + 1).start()
        return carry

    lax.fori_loop(0, n // NBUF, group, 0)

    for b in range(NBUF):
        _wb_desc(bufs[b], out_hbm, wsems[b],
                 seq_base + n - NBUF + b).wait()


def kernel(inputs, token_table, position_table):
    mesh = plsc.VectorSubcoreMesh(core_axis_name="c", subcore_axis_name="s")
    run = functools.partial(
        pl.kernel,
        out_type=jax.ShapeDtypeStruct((BATCH, SEQ, EMBED_DIM), jnp.float32),
        mesh=mesh,
        scratch_types=(
            [pltpu.VMEM((CHUNKS_PER_WORKER, SEQ), jnp.int32),
             pltpu.VMEM((SEQ, EMBED_DIM), jnp.float32)]
            + [pltpu.VMEM((SEQ, EMBED_DIM), jnp.float32) for _ in range(NBUF)]
            + [pltpu.SemaphoreType.DMA for _ in range(2 * NBUF)]
        ),
        compiler_params=pltpu.CompilerParams(use_tc_tiling_on_sc=False),
    )(_tpe_kernel)
    return run(inputs.astype(jnp.int32), token_table, position_table)
